# Initial kernel scaffold; baseline (speedup 1.0000x reference)
import jax
import jax.numpy as jnp
from jax.experimental import pallas as pl


def kernel(desc_A, desc_B, matchability_A, matchability_B):
    raise NotImplementedError("write your pallas kernel here")


# speedup vs baseline: 1.5409x; 1.5409x over previous
import functools

import jax
import jax.numpy as jnp
from jax import lax
from jax.experimental import pallas as pl
from jax.experimental.pallas import tpu as pltpu

_THRESH = -3.0
_BM = 256


def _pass1_body(dA_ref, dB_ref, mA_ref, mB_ref,
                scores_ref, rmax_ref, rarg_ref, carg_ref,
                dBn_scr, cmax_scr, cidx_scr, *, nm, precision):
    i = pl.program_id(1)

    @pl.when(i == 0)
    def _():
        dB = dB_ref[0]
        nB = jnp.sqrt(jnp.sum(dB * dB, axis=-1, keepdims=True))
        dBn_scr[...] = dB / (nB + 1e-12)

    dA = dA_ref[0]
    nA = jnp.sqrt(jnp.sum(dA * dA, axis=-1, keepdims=True))
    dAn = dA / (nA + 1e-12)
    d = dA.shape[-1]
    s = lax.dot_general(dAn, dBn_scr[...], (((1,), (1,)), ((), ())),
                        precision=precision,
                        preferred_element_type=jnp.float32)
    s = s * (float(d) ** 0.5)
    s = (s + mA_ref[0, 0][:, None]) + mB_ref[0, 0][None, :]
    scores_ref[0] = s

    rmax_ref[0, 0] = jnp.max(s, axis=1)
    rarg_ref[0, 0] = jnp.argmax(s, axis=1).astype(jnp.int32)

    bm = s.shape[0]
    tmax = jnp.max(s, axis=0, keepdims=True)
    targ = jnp.argmax(s, axis=0).astype(jnp.int32)[None, :] + i * bm
    prev_max = cmax_scr[...]
    prev_arg = cidx_scr[...]
    upd = jnp.logical_or(i == 0, tmax > prev_max)
    cmax_scr[...] = jnp.where(upd, tmax, prev_max)
    cidx_scr[...] = jnp.where(upd, targ, prev_arg)

    @pl.when(i == nm - 1)
    def _():
        carg_ref[0] = cidx_scr[...]


def _pass2_body(rmax_ref, rarg_ref, carg_ref, conf_ref):
    i = pl.program_id(1)
    rm = rmax_ref[0, 0]
    ra = rarg_ref[0, 0]
    ca = carg_ref[0, 0]
    bm = rm.shape[0]
    n = ca.shape[0]
    col_iota = lax.broadcasted_iota(jnp.int32, (bm, n), 1)
    row_iota = lax.broadcasted_iota(jnp.int32, (bm, n), 0) + i * bm
    mut = jnp.logical_and(
        jnp.logical_and(col_iota == ra[:, None], ca[None, :] == row_iota),
        rm[:, None] >= _THRESH)
    sig = jax.nn.sigmoid(rm)
    conf_ref[0] = jnp.where(mut, jnp.broadcast_to(sig[:, None], (bm, n)), 0.0)


def kernel(desc_A, desc_B, matchability_A, matchability_B):
    B, M, D = desc_A.shape
    N = desc_B.shape[1]
    bm = _BM
    nm = M // bm
    mA3 = matchability_A.reshape(B, 1, M)
    mB3 = matchability_B.reshape(B, 1, N)

    p1 = pl.pallas_call(
        functools.partial(_pass1_body, nm=nm, precision=lax.Precision.DEFAULT),
        grid=(B, nm),
        in_specs=[
            pl.BlockSpec((1, bm, D), lambda b, i: (b, i, 0)),
            pl.BlockSpec((1, N, D), lambda b, i: (b, 0, 0)),
            pl.BlockSpec((1, 1, bm), lambda b, i: (b, 0, i)),
            pl.BlockSpec((1, 1, N), lambda b, i: (b, 0, 0)),
        ],
        out_specs=[
            pl.BlockSpec((1, bm, N), lambda b, i: (b, i, 0)),
            pl.BlockSpec((1, 1, bm), lambda b, i: (b, 0, i)),
            pl.BlockSpec((1, 1, bm), lambda b, i: (b, 0, i)),
            pl.BlockSpec((1, 1, N), lambda b, i: (b, 0, 0)),
        ],
        out_shape=[
            jax.ShapeDtypeStruct((B, M, N), jnp.float32),
            jax.ShapeDtypeStruct((B, 1, M), jnp.float32),
            jax.ShapeDtypeStruct((B, 1, M), jnp.int32),
            jax.ShapeDtypeStruct((B, 1, N), jnp.int32),
        ],
        scratch_shapes=[
            pltpu.VMEM((N, D), jnp.float32),
            pltpu.VMEM((1, N), jnp.float32),
            pltpu.VMEM((1, N), jnp.int32),
        ],
        compiler_params=pltpu.CompilerParams(
            dimension_semantics=("arbitrary", "arbitrary")),
    )
    scores, rowmax, rowarg, colarg = p1(desc_A, desc_B, mA3, mB3)

    p2 = pl.pallas_call(
        _pass2_body,
        grid=(B, nm),
        in_specs=[
            pl.BlockSpec((1, 1, bm), lambda b, i: (b, 0, i)),
            pl.BlockSpec((1, 1, bm), lambda b, i: (b, 0, i)),
            pl.BlockSpec((1, 1, N), lambda b, i: (b, 0, 0)),
        ],
        out_specs=pl.BlockSpec((1, bm, N), lambda b, i: (b, i, 0)),
        out_shape=jax.ShapeDtypeStruct((B, M, N), jnp.float32),
        compiler_params=pltpu.CompilerParams(
            dimension_semantics=("arbitrary", "arbitrary")),
    )
    confidence = p2(rowmax, rowarg, colarg)
    return scores, confidence
